# baseline (speedup 1.0000x reference)
import jax
import jax.numpy as jnp
from jax import lax
from jax.experimental import pallas as pl
from jax.experimental.pallas import tpu as pltpu
from jax.experimental.pallas import tpu_sc as plsc

NUM_USERS = 25000
N = 50000
H = 32
E = 800000
LAYERS = 3

NC = 2
NS = 16
C = 256
G = 128
CG = C // G
CH = 198
EPT = CH * C
E_PAD = EPT * NS
TOTCH = E_PAD // C
NP = 50176
RPT = NP // NS
RC = 112
WB = RPT // RC
ZR = 56


def _gcn_body(t0, meta, val, mean_o, ta, tb,
              ms0, ms1, ms2, vs0, vs1, vs2, rs0, rs1, zb_v, acc,
              gsem0, gsem1, msem0, msem1, ssem):
    c = lax.axis_index("c")
    s = lax.axis_index("s")
    ms = [ms0, ms1, ms2]
    vs = [vs0, vs1, vs2]
    rs = [rs0, rs1]
    gsems = [gsem0, gsem1]
    msems = [msem0, msem1]

    zero16 = jnp.zeros((16,), jnp.float32)

    def zb_zero(r, carry):
        zb_v[r, pl.ds(0, 16)] = zero16
        zb_v[r, pl.ds(16, 16)] = zero16
        return carry
    lax.fori_loop(0, ZR, zb_zero, 0)

    def acc_zero(k, carry):
        pltpu.sync_copy(zb_v, acc.at[pl.ds(s * RPT + k * ZR, ZR)])
        return carry
    lax.fori_loop(0, RPT // ZR, acc_zero, 0)
    plsc.subcore_barrier()

    def meta_src(i):
        return meta.at[c, pl.ds((s * CH + i) * 4, 4)]

    def val_src(i):
        return val.at[pl.ds((s * CH + i) * C, C)]

    def fire_gathers(tbl, m, rbuf, sem):
        for j in range(CG):
            pltpu.async_copy(tbl.at[m.at[j]], rbuf.at[pl.ds(j * G, G)], sem)

    def wait_gathers(tbl, m, rbuf, sem):
        for j in range(CG):
            pltpu.make_async_copy(tbl.at[m.at[j]],
                                  rbuf.at[pl.ds(j * G, G)], sem).wait()

    def fire_scatters(rbuf, m, sem):
        for j in range(CG):
            pltpu.async_copy(rbuf.at[pl.ds(j * G, G)],
                             acc.at[m.at[CG + j]], sem, add=True)

    def wait_scatters(rbuf, m, sem):
        for j in range(CG):
            pltpu.make_async_copy(rbuf.at[pl.ds(j * G, G)],
                                  acc.at[m.at[CG + j]], sem).wait()

    def mul_chunk(rbuf, vbuf):
        def mul_body(g, mcarry):
            vv = vbuf[pl.ds(g * 16, 16)]
            for t in range(16):
                e = g * 16 + t
                vt = vv[t]
                rbuf[e, pl.ds(0, 16)] = rbuf[e, pl.ds(0, 16)] * vt
                rbuf[e, pl.ds(16, 16)] = rbuf[e, pl.ds(16, 16)] * vt
            return mcarry
        lax.fori_loop(0, C // 16, mul_body, 0)

    tables = [t0, ta, tb]
    for layer in range(LAYERS):
        src_table = tables[layer]
        dst_table = tables[layer + 1] if layer < LAYERS - 1 else None

        pltpu.sync_copy(meta_src(0), ms[0])
        pltpu.sync_copy(val_src(0), vs[0])
        pltpu.async_copy(meta_src(1), ms[1], msems[1])
        pltpu.async_copy(val_src(1), vs[1], msems[1])
        fire_gathers(src_table, ms[0], rs[0], gsems[0])

        def ring_body(i2, carry):
            for b in range(6):
                i = i2 * 6 + b
                slot_c, rows_c = b % 3, b % 2
                slot_p = (b + 2) % 3
                slot_n, rows_n = (b + 1) % 3, (b + 1) % 2

                if b == 0:
                    @pl.when(i2 > 0)
                    def _():
                        wait_scatters(rs[rows_n], ms[slot_p], ssem)
                else:
                    wait_scatters(rs[rows_n], ms[slot_p], ssem)

                def fire_m():
                    pltpu.async_copy(meta_src(i + 2), ms[slot_p],
                                     msems[b % 2])
                    pltpu.async_copy(val_src(i + 2), vs[slot_p],
                                     msems[b % 2])
                if b < 4:
                    fire_m()
                else:
                    pl.when(i2 < CH // 6 - 1)(fire_m)

                def fire_g():
                    pltpu.make_async_copy(meta_src(i + 1), ms[slot_n],
                                          msems[(b + 1) % 2]).wait()
                    pltpu.make_async_copy(val_src(i + 1), vs[slot_n],
                                          msems[(b + 1) % 2]).wait()
                    fire_gathers(src_table, ms[slot_n], rs[rows_n],
                                 gsems[rows_n])
                if b < 5:
                    fire_g()
                else:
                    pl.when(i2 < CH // 6 - 1)(fire_g)

                wait_gathers(src_table, ms[slot_c], rs[rows_c],
                             gsems[rows_c])
                mul_chunk(rs[rows_c], vs[slot_c])
                fire_scatters(rs[rows_c], ms[slot_c], ssem)
            return carry
        lax.fori_loop(0, CH // 6, ring_body, 0)
        wait_scatters(rs[(CH - 1) % 2], ms[(CH - 1) % 3], ssem)
        plsc.subcore_barrier()

        last = layer == LAYERS - 1
        scale = 1.0 / (LAYERS + 1) if last else 1.0
        def wb_row0(k):
            return s * RPT + k * RC

        def mean_ref(k):
            return mean_o.at[pl.ds(wb_row0(k), RC), pl.ds(c * H, H)]

        def mref_src(k):
            if layer == 0:
                return t0.at[pl.ds(c * NP + wb_row0(k), RC)]
            return mean_ref(k)

        def fire_wb_reads(k, buf, sem):
            pltpu.async_copy(mref_src(k), buf.at[pl.ds(RC, RC)], sem)

        def wait_wb_reads(k, buf, sem):
            pltpu.make_async_copy(mref_src(k),
                                  buf.at[pl.ds(RC, RC)], sem).wait()

        def fire_wb_writes(k, buf, sem):
            if dst_table is not None:
                pltpu.async_copy(buf.at[pl.ds(0, RC)],
                                 dst_table.at[pl.ds(c * NP + wb_row0(k), RC)],
                                 sem)
            pltpu.async_copy(buf.at[pl.ds(RC, RC)], mean_ref(k), sem)

        def wait_wb_writes(k, buf, sem):
            if dst_table is not None:
                pltpu.make_async_copy(
                    buf.at[pl.ds(0, RC)],
                    dst_table.at[pl.ds(c * NP + wb_row0(k), RC)], sem).wait()
            pltpu.make_async_copy(buf.at[pl.ds(RC, RC)],
                                  mean_ref(k), sem).wait()

        fire_wb_reads(0, rs[0], gsems[0])

        def wb_ring(k2, carry):
            for b in range(2):
                k = k2 * 2 + b
                if b == 0:
                    @pl.when(k2 > 0)
                    def _():
                        wait_wb_writes(k - 1, rs[1 - b], msems[1 - b])
                else:
                    wait_wb_writes(k - 1, rs[1 - b], msems[1 - b])

                def fire_r():
                    fire_wb_reads(k + 1, rs[1 - b], gsems[1 - b])
                if b == 0:
                    fire_r()
                else:
                    pl.when(k2 < WB // 2 - 1)(fire_r)

                pltpu.sync_copy(acc.at[pl.ds(wb_row0(k), RC)],
                                rs[b].at[pl.ds(0, RC)])
                wait_wb_reads(k, rs[b], gsems[b])
                buf = rs[b]

                def add_body(r, acarry):
                    a0 = (buf[RC + r, pl.ds(0, 16)]
                          + buf[r, pl.ds(0, 16)]) * scale
                    a1 = (buf[RC + r, pl.ds(16, 16)]
                          + buf[r, pl.ds(16, 16)]) * scale
                    buf[RC + r, pl.ds(0, 16)] = a0
                    buf[RC + r, pl.ds(16, 16)] = a1
                    return acarry
                lax.fori_loop(0, RC, add_body, 0)
                fire_wb_writes(k, rs[b], msems[b])
                if not last:
                    pltpu.sync_copy(zb_v, acc.at[pl.ds(wb_row0(k), ZR)])
                    pltpu.sync_copy(zb_v, acc.at[pl.ds(wb_row0(k) + ZR, ZR)])
            return carry
        lax.fori_loop(0, WB // 2, wb_ring, 0)
        wait_wb_writes(WB - 1, rs[1], msems[1])
        if not last:
            plsc.subcore_barrier()


@jax.jit
def _gcn(t0, meta, val):
    mesh = plsc.VectorSubcoreMesh(core_axis_name="c", subcore_axis_name="s",
                                  num_cores=NC, num_subcores=NS)
    f = pl.kernel(
        _gcn_body,
        out_type=jax.ShapeDtypeStruct((NP, 2 * H), jnp.float32),
        mesh=mesh,
        scratch_types=[
            pltpu.HBM((2 * NP, H), jnp.float32),
            pltpu.HBM((2 * NP, H), jnp.float32),
            pltpu.VMEM((4, G), jnp.int32),
            pltpu.VMEM((4, G), jnp.int32),
            pltpu.VMEM((4, G), jnp.int32),
            pltpu.VMEM((C,), jnp.float32),
            pltpu.VMEM((C,), jnp.float32),
            pltpu.VMEM((C,), jnp.float32),
            pltpu.VMEM((C, H), jnp.float32),
            pltpu.VMEM((C, H), jnp.float32),
            pltpu.VMEM((ZR, H), jnp.float32),
            pltpu.VMEM_SHARED((NP, H), jnp.float32),
            pltpu.SemaphoreType.DMA,
            pltpu.SemaphoreType.DMA,
            pltpu.SemaphoreType.DMA,
            pltpu.SemaphoreType.DMA,
            pltpu.SemaphoreType.DMA,
        ],
        compiler_params=pltpu.CompilerParams(use_tc_tiling_on_sc=False),
    )
    return f(t0, meta, val)


def kernel(embeddings, edge_values, edge_index):
    src = edge_index[0].astype(jnp.int32)
    dst = edge_index[1].astype(jnp.int32)
    pad = E_PAD - E
    src_p = jnp.concatenate([src, jnp.zeros((pad,), jnp.int32)])
    dst_p = jnp.concatenate([dst, jnp.zeros((pad,), jnp.int32)])
    val_p = jnp.concatenate([edge_values.astype(jnp.float32),
                             jnp.zeros((pad,), jnp.float32)])
    srcs = jnp.stack([src_p, src_p + NP]).reshape(NC, TOTCH, CG, G)
    dsts = jnp.broadcast_to(dst_p.reshape(TOTCH, CG, G), (NC, TOTCH, CG, G))
    meta = jnp.concatenate([srcs, dsts], axis=2).reshape(NC, TOTCH * 4, G)
    rpad = jnp.zeros((NP - N, H), jnp.float32)
    t0 = jnp.concatenate(
        [embeddings[:, :H], rpad, embeddings[:, H:], rpad], axis=0)
    out = _gcn(t0, meta, val_p)[:N]
    return out[:NUM_USERS], out[NUM_USERS:]


# speedup vs baseline: 1.0577x; 1.0577x over previous
import jax
import jax.numpy as jnp
from jax import lax
from jax.experimental import pallas as pl
from jax.experimental.pallas import tpu as pltpu
from jax.experimental.pallas import tpu_sc as plsc

NUM_USERS = 25000
N = 50000
H = 32
E = 800000
LAYERS = 3

NC = 2
NS = 16
C = 256
G = 128
CG = C // G
CH = 198
EPT = CH * C
E_PAD = EPT * NS
TOTCH = E_PAD // C
NP = 50176
RPT = NP // NS
RC = 112
WB = RPT // RC
ZR = 56


def _gcn_body(t0, meta, val, mean_o, ta, tb,
              ms0, ms1, ms2, vs0, vs1, vs2, rs0, rs1, zb_v, acc,
              gsem0, gsem1, msem0, msem1, ssem):
    c = lax.axis_index("c")
    s = lax.axis_index("s")
    ms = [ms0, ms1, ms2]
    vs = [vs0, vs1, vs2]
    rs = [rs0, rs1]
    gsems = [gsem0, gsem1]
    msems = [msem0, msem1]

    zero16 = jnp.zeros((16,), jnp.float32)

    def zb_zero(r, carry):
        zb_v[r, pl.ds(0, 16)] = zero16
        zb_v[r, pl.ds(16, 16)] = zero16
        return carry
    lax.fori_loop(0, ZR, zb_zero, 0)

    def acc_zero(k, carry):
        pltpu.sync_copy(zb_v, acc.at[pl.ds(s * RPT + k * ZR, ZR)])
        return carry
    lax.fori_loop(0, RPT // ZR, acc_zero, 0)
    plsc.subcore_barrier()

    def meta_src(i):
        return meta.at[c, pl.ds((s * CH + i) * 4, 4)]

    def val_src(i):
        return val.at[pl.ds((s * CH + i) * C, C)]

    def fire_gathers(tbl, m, rbuf, sem):
        for j in range(CG):
            pltpu.async_copy(tbl.at[m.at[j]], rbuf.at[pl.ds(j * G, G)], sem)

    def wait_gathers(tbl, m, rbuf, sem):
        for j in range(CG):
            pltpu.make_async_copy(tbl.at[m.at[j]],
                                  rbuf.at[pl.ds(j * G, G)], sem).wait()

    def fire_scatters(rbuf, m, sem):
        for j in range(CG):
            pltpu.async_copy(rbuf.at[pl.ds(j * G, G)],
                             acc.at[m.at[CG + j]], sem, add=True)

    def wait_scatters(rbuf, m, sem):
        for j in range(CG):
            pltpu.make_async_copy(rbuf.at[pl.ds(j * G, G)],
                                  acc.at[m.at[CG + j]], sem).wait()

    def mul_chunk(rbuf, vbuf):
        def mul_body(g, mcarry):
            vv = vbuf[pl.ds(g * 16, 16)]
            for t in range(16):
                e = g * 16 + t
                vt = vv[t]
                rbuf[e, pl.ds(0, 16)] = rbuf[e, pl.ds(0, 16)] * vt
                rbuf[e, pl.ds(16, 16)] = rbuf[e, pl.ds(16, 16)] * vt
            return mcarry
        lax.fori_loop(0, C // 16, mul_body, 0)

    tables = [t0, ta, tb]
    for layer in range(LAYERS):
        src_table = tables[layer]
        dst_table = tables[layer + 1] if layer < LAYERS - 1 else None

        pltpu.sync_copy(meta_src(0), ms[0])
        pltpu.sync_copy(val_src(0), vs[0])
        pltpu.async_copy(meta_src(1), ms[1], msems[1])
        pltpu.async_copy(val_src(1), vs[1], msems[1])
        fire_gathers(src_table, ms[0], rs[0], gsems[0])

        def ring_body(i2, carry):
            for b in range(6):
                i = i2 * 6 + b
                slot_c, rows_c = b % 3, b % 2
                slot_p = (b + 2) % 3
                slot_n, rows_n = (b + 1) % 3, (b + 1) % 2

                if b == 0:
                    @pl.when(i2 > 0)
                    def _():
                        wait_scatters(rs[rows_n], ms[slot_p], ssem)
                else:
                    wait_scatters(rs[rows_n], ms[slot_p], ssem)

                def fire_m():
                    pltpu.async_copy(meta_src(i + 2), ms[slot_p],
                                     msems[b % 2])
                    pltpu.async_copy(val_src(i + 2), vs[slot_p],
                                     msems[b % 2])
                if b < 4:
                    fire_m()
                else:
                    pl.when(i2 < CH // 6 - 1)(fire_m)

                def fire_g():
                    pltpu.make_async_copy(meta_src(i + 1), ms[slot_n],
                                          msems[(b + 1) % 2]).wait()
                    pltpu.make_async_copy(val_src(i + 1), vs[slot_n],
                                          msems[(b + 1) % 2]).wait()
                    fire_gathers(src_table, ms[slot_n], rs[rows_n],
                                 gsems[rows_n])
                if b < 5:
                    fire_g()
                else:
                    pl.when(i2 < CH // 6 - 1)(fire_g)

                wait_gathers(src_table, ms[slot_c], rs[rows_c],
                             gsems[rows_c])
                mul_chunk(rs[rows_c], vs[slot_c])
                fire_scatters(rs[rows_c], ms[slot_c], ssem)
            return carry
        lax.fori_loop(0, CH // 6, ring_body, 0)
        wait_scatters(rs[(CH - 1) % 2], ms[(CH - 1) % 3], ssem)
        plsc.subcore_barrier()

        last = layer == LAYERS - 1
        scale = 1.0 / (LAYERS + 1) if last else 1.0
        mref = t0 if layer == 0 else mean_o

        def wb_row0(k):
            return s * RPT + k * RC

        def fire_wb_reads(k, buf, sem):
            pltpu.async_copy(mref.at[pl.ds(c * NP + wb_row0(k), RC)],
                             buf.at[pl.ds(RC, RC)], sem)

        def wait_wb_reads(k, buf, sem):
            pltpu.make_async_copy(mref.at[pl.ds(c * NP + wb_row0(k), RC)],
                                  buf.at[pl.ds(RC, RC)], sem).wait()

        def fire_wb_writes(k, buf, sem):
            if dst_table is not None:
                pltpu.async_copy(buf.at[pl.ds(0, RC)],
                                 dst_table.at[pl.ds(c * NP + wb_row0(k), RC)],
                                 sem)
            pltpu.async_copy(buf.at[pl.ds(RC, RC)],
                             mean_o.at[pl.ds(c * NP + wb_row0(k), RC)], sem)

        def wait_wb_writes(k, buf, sem):
            if dst_table is not None:
                pltpu.make_async_copy(
                    buf.at[pl.ds(0, RC)],
                    dst_table.at[pl.ds(c * NP + wb_row0(k), RC)], sem).wait()
            pltpu.make_async_copy(
                buf.at[pl.ds(RC, RC)],
                mean_o.at[pl.ds(c * NP + wb_row0(k), RC)], sem).wait()

        fire_wb_reads(0, rs[0], gsems[0])

        def wb_ring(k2, carry):
            for b in range(2):
                k = k2 * 2 + b
                if b == 0:
                    @pl.when(k2 > 0)
                    def _():
                        wait_wb_writes(k - 1, rs[1 - b], msems[1 - b])
                else:
                    wait_wb_writes(k - 1, rs[1 - b], msems[1 - b])

                def fire_r():
                    fire_wb_reads(k + 1, rs[1 - b], gsems[1 - b])
                if b == 0:
                    fire_r()
                else:
                    pl.when(k2 < WB // 2 - 1)(fire_r)

                pltpu.sync_copy(acc.at[pl.ds(wb_row0(k), RC)],
                                rs[b].at[pl.ds(0, RC)])
                wait_wb_reads(k, rs[b], gsems[b])
                buf = rs[b]

                def add_body(r, acarry):
                    a0 = (buf[RC + r, pl.ds(0, 16)]
                          + buf[r, pl.ds(0, 16)]) * scale
                    a1 = (buf[RC + r, pl.ds(16, 16)]
                          + buf[r, pl.ds(16, 16)]) * scale
                    buf[RC + r, pl.ds(0, 16)] = a0
                    buf[RC + r, pl.ds(16, 16)] = a1
                    return acarry
                lax.fori_loop(0, RC, add_body, 0)
                fire_wb_writes(k, rs[b], msems[b])
                if not last:
                    pltpu.sync_copy(zb_v, acc.at[pl.ds(wb_row0(k), ZR)])
                    pltpu.sync_copy(zb_v, acc.at[pl.ds(wb_row0(k) + ZR, ZR)])
            return carry
        lax.fori_loop(0, WB // 2, wb_ring, 0)
        wait_wb_writes(WB - 1, rs[1], msems[1])
        if not last:
            plsc.subcore_barrier()


@jax.jit
def _gcn(t0, meta, val):
    mesh = plsc.VectorSubcoreMesh(core_axis_name="c", subcore_axis_name="s",
                                  num_cores=NC, num_subcores=NS)
    f = pl.kernel(
        _gcn_body,
        out_type=jax.ShapeDtypeStruct((2 * NP, H), jnp.float32),
        mesh=mesh,
        scratch_types=[
            pltpu.HBM((2 * NP, H), jnp.float32),
            pltpu.HBM((2 * NP, H), jnp.float32),
            pltpu.VMEM((4, G), jnp.int32),
            pltpu.VMEM((4, G), jnp.int32),
            pltpu.VMEM((4, G), jnp.int32),
            pltpu.VMEM((C,), jnp.float32),
            pltpu.VMEM((C,), jnp.float32),
            pltpu.VMEM((C,), jnp.float32),
            pltpu.VMEM((C, H), jnp.float32),
            pltpu.VMEM((C, H), jnp.float32),
            pltpu.VMEM((ZR, H), jnp.float32),
            pltpu.VMEM_SHARED((NP, H), jnp.float32),
            pltpu.SemaphoreType.DMA,
            pltpu.SemaphoreType.DMA,
            pltpu.SemaphoreType.DMA,
            pltpu.SemaphoreType.DMA,
            pltpu.SemaphoreType.DMA,
        ],
        compiler_params=pltpu.CompilerParams(use_tc_tiling_on_sc=False),
    )
    return f(t0, meta, val)


def kernel(embeddings, edge_values, edge_index):
    src = edge_index[0].astype(jnp.int32)
    dst = edge_index[1].astype(jnp.int32)
    pad = E_PAD - E
    src_p = jnp.concatenate([src, jnp.zeros((pad,), jnp.int32)])
    dst_p = jnp.concatenate([dst, jnp.zeros((pad,), jnp.int32)])
    val_p = jnp.concatenate([edge_values.astype(jnp.float32),
                             jnp.zeros((pad,), jnp.float32)])
    srcs = jnp.stack([src_p, src_p + NP]).reshape(NC, TOTCH, CG, G)
    dsts = jnp.broadcast_to(dst_p.reshape(TOTCH, CG, G), (NC, TOTCH, CG, G))
    meta = jnp.concatenate([srcs, dsts], axis=2).reshape(NC, TOTCH * 4, G)
    rpad = jnp.zeros((NP - N, H), jnp.float32)
    t0 = jnp.concatenate(
        [embeddings[:, :H], rpad, embeddings[:, H:], rpad], axis=0)
    mean = _gcn(t0, meta, val_p)
    out = jnp.concatenate([mean[:N], mean[NP:NP + N]], axis=1)
    return out[:NUM_USERS], out[NUM_USERS:]
